# baseline (speedup 1.0000x reference)
import functools

import jax
import jax.numpy as jnp
from jax import lax
from jax.experimental import pallas as pl
from jax.experimental.pallas import tpu as pltpu
from jax.experimental.pallas import tpu_sc as plsc

N = 10000
E = 320000
NC, NS = 2, 16
NW = NC * NS
E_PAD = 327680
EPW = E_PAD // NW
CG = 320
NG = EPW // CG
CS = 128
NCH = EPW // CS
N_PAD = 10240
NPT = N_PAD // NS
BE = 512
BN = 2048
NB = N_PAD // BN

_HIGH = jax.lax.Precision.HIGHEST


def _mesh():
    return plsc.VectorSubcoreMesh(core_axis_name="c", subcore_axis_name="s")


def _gather_body(prow_h, pcol_h, row_h, col_h, grow_h, gcol_h,
                 idx_r, buf_r, idx_c, buf_c, sem_r, sem_c):
    wid = lax.axis_index("s") * NC + lax.axis_index("c")
    base = wid * EPW

    def body(j, carry):
        off = base + j * CG
        pltpu.sync_copy(row_h.at[pl.ds(off, CG)], idx_r)
        pltpu.sync_copy(col_h.at[pl.ds(off, CG)], idx_c)
        cr = pltpu.async_copy(prow_h.at[idx_r], buf_r, sem_r)
        cc = pltpu.async_copy(pcol_h.at[idx_c], buf_c, sem_c)
        cr.wait()
        pltpu.sync_copy(buf_r, grow_h.at[pl.ds(off, CG)])
        cc.wait()
        pltpu.sync_copy(buf_c, gcol_h.at[pl.ds(off, CG)])
        return carry

    lax.fori_loop(0, NG, body, 0)


def _sc_gather(prow, pcol, row, col):
    return pl.kernel(
        _gather_body,
        out_type=[jax.ShapeDtypeStruct((E_PAD, 128), jnp.float32),
                  jax.ShapeDtypeStruct((E_PAD, 128), jnp.float32)],
        mesh=_mesh(),
        scratch_types=[
            pltpu.VMEM((CG,), jnp.int32),
            pltpu.VMEM((CG, 128), jnp.float32),
            pltpu.VMEM((CG,), jnp.int32),
            pltpu.VMEM((CG, 128), jnp.float32),
            pltpu.SemaphoreType.DMA,
            pltpu.SemaphoreType.DMA,
        ],
    )(prow, pcol, row, col)


def _scatter_body(m_h, col_h, zero_h, sums_h, idx_a, idx_b, vals_a, vals_b,
                  acc_s):
    ci = lax.axis_index("c")
    si = lax.axis_index("s")
    wid = si * NC + ci
    pltpu.sync_copy(zero_h.at[pl.ds(si * NPT, NPT)],
                    acc_s.at[pl.ds(si * NPT, NPT)])
    plsc.subcore_barrier()

    def body(k, carry):
        for ph, ibuf, vbuf in ((0, idx_a, vals_a), (1, idx_b, vals_b)):
            j = 2 * k + ph
            off = wid * EPW + j * CS
            pltpu.sync_copy(col_h.at[pl.ds(off, CS)], ibuf)
            pltpu.sync_copy(m_h.at[pl.ds(off, CS)], vbuf)
            pltpu.sync_copy(vbuf, acc_s.at[ibuf], add=True)
        return carry

    lax.fori_loop(0, NCH // 2, body, 0)
    plsc.subcore_barrier()
    pltpu.sync_copy(acc_s.at[pl.ds(si * NPT, NPT)],
                    sums_h.at[pl.ds(ci * N_PAD + si * NPT, NPT)])


def _sc_scatter(m, colp, zeros64):
    return pl.kernel(
        _scatter_body,
        out_type=jax.ShapeDtypeStruct((NC * N_PAD, 64), jnp.float32),
        mesh=_mesh(),
        scratch_types=[
            pltpu.VMEM((CS,), jnp.int32),
            pltpu.VMEM((CS,), jnp.int32),
            pltpu.VMEM((CS, 64), jnp.float32),
            pltpu.VMEM((CS, 64), jnp.float32),
            pltpu.VMEM_SHARED((N_PAD, 64), jnp.float32),
        ],
    )(m, colp, zeros64)


def _count_body(col_h, zero_h, ones_h, cnt_h, idx_a, idx_b, ones_v, acc_s):
    ci = lax.axis_index("c")
    si = lax.axis_index("s")
    wid = si * NC + ci
    pltpu.sync_copy(zero_h.at[pl.ds(si * NPT, NPT)],
                    acc_s.at[pl.ds(si * NPT, NPT)])
    pltpu.sync_copy(ones_h, ones_v)
    plsc.subcore_barrier()

    def body(k, carry):
        for ph, ibuf in ((0, idx_a), (1, idx_b)):
            j = 2 * k + ph
            off = wid * EPW + j * CS
            pltpu.sync_copy(col_h.at[pl.ds(off, CS)], ibuf)
            pltpu.sync_copy(ones_v, acc_s.at[ibuf], add=True)
        return carry

    lax.fori_loop(0, NCH // 2, body, 0)
    plsc.subcore_barrier()
    pltpu.sync_copy(acc_s.at[pl.ds(si * NPT, NPT)],
                    cnt_h.at[pl.ds(ci * N_PAD + si * NPT, NPT)])


def _sc_count(colp, zeros64, ones64):
    return pl.kernel(
        _count_body,
        out_type=jax.ShapeDtypeStruct((NC * N_PAD, 64), jnp.float32),
        mesh=_mesh(),
        scratch_types=[
            pltpu.VMEM((CS,), jnp.int32),
            pltpu.VMEM((CS,), jnp.int32),
            pltpu.VMEM((CS, 64), jnp.float32),
            pltpu.VMEM_SHARED((N_PAD, 64), jnp.float32),
        ],
    )(colp, zeros64, ones64)


def _edge_body(last, g_ref, gc_ref, ea_ref, wee_ref, w1e_ref,
               be_ref, b1_ref, e_ref, m_ref):
    g = g_ref[...]
    e = (g[:, :64] + gc_ref[...][:, :64]
         + jnp.dot(ea_ref[...], wee_ref[...], precision=_HIGH,
                   preferred_element_type=jnp.float32)
         + be_ref[...])
    if not last:
        e = jnp.maximum(e, 0.0)
    e_ref[...] = e
    m = (g[:, 64:]
         + jnp.dot(e, w1e_ref[...], precision=_HIGH,
                   preferred_element_type=jnp.float32)
         + b1_ref[...])
    if not last:
        m = jnp.maximum(m, 0.0)
    m_ref[...] = m


def _edge_mlp(grow, gcol, ea, wee, w1e, be_, b1, last):
    de = ea.shape[1]
    return pl.pallas_call(
        functools.partial(_edge_body, last),
        grid=(E_PAD // BE,),
        in_specs=[
            pl.BlockSpec((BE, 128), lambda i: (i, 0)),
            pl.BlockSpec((BE, 128), lambda i: (i, 0)),
            pl.BlockSpec((BE, de), lambda i: (i, 0)),
            pl.BlockSpec((de, 64), lambda i: (0, 0)),
            pl.BlockSpec((64, 64), lambda i: (0, 0)),
            pl.BlockSpec((1, 64), lambda i: (0, 0)),
            pl.BlockSpec((1, 64), lambda i: (0, 0)),
        ],
        out_specs=[pl.BlockSpec((BE, 64), lambda i: (i, 0)),
                   pl.BlockSpec((BE, 64), lambda i: (i, 0))],
        out_shape=[jax.ShapeDtypeStruct((E_PAD, 64), jnp.float32),
                   jax.ShapeDtypeStruct((E_PAD, 64), jnp.float32)],
    )(grow, gcol, ea, wee, w1e, be_, b1)


def _node_body(last, x_ref, s0_ref, s1_ref, c0_ref, c1_ref, w2x_ref,
               w2a_ref, b2_ref, *rest):
    cnt = c0_ref[...][:, 0:1] + c1_ref[...][:, 0:1]
    recip = 1.0 / jnp.maximum(cnt, 1.0)
    agg = (s0_ref[...] + s1_ref[...]) * recip
    h = (jnp.dot(x_ref[...], w2x_ref[...], precision=_HIGH,
                 preferred_element_type=jnp.float32)
         + jnp.dot(agg, w2a_ref[...], precision=_HIGH,
                   preferred_element_type=jnp.float32)
         + b2_ref[...])
    if not last:
        h = jnp.maximum(h, 0.0)
    if last:
        (xn_ref,) = rest
        xn_ref[...] = h
    else:
        wrow_ref, wcol_ref, xn_ref, prow_ref, pcol_ref = rest
        xn_ref[...] = h
        prow_ref[...] = jnp.dot(h, wrow_ref[...], precision=_HIGH,
                                preferred_element_type=jnp.float32)
        pcol_ref[...] = jnp.dot(h, wcol_ref[...], precision=_HIGH,
                                preferred_element_type=jnp.float32)


def _node_update(x, sums, cnt, w2x, w2a, b2, wrow_n, wcol_n, last):
    dx = x.shape[1]
    in_specs = [
        pl.BlockSpec((BN, dx), lambda i: (i, 0)),
        pl.BlockSpec((BN, 64), lambda i: (i, 0)),
        pl.BlockSpec((BN, 64), lambda i: (i + NB, 0)),
        pl.BlockSpec((BN, 64), lambda i: (i, 0)),
        pl.BlockSpec((BN, 64), lambda i: (i + NB, 0)),
        pl.BlockSpec((dx, 64), lambda i: (0, 0)),
        pl.BlockSpec((64, 64), lambda i: (0, 0)),
        pl.BlockSpec((1, 64), lambda i: (0, 0)),
    ]
    args = [x, sums, sums, cnt, cnt, w2x, w2a, b2]
    if last:
        out_specs = pl.BlockSpec((BN, 64), lambda i: (i, 0))
        out_shape = jax.ShapeDtypeStruct((N, 64), jnp.float32)
    else:
        in_specs += [pl.BlockSpec((64, 128), lambda i: (0, 0)),
                     pl.BlockSpec((64, 128), lambda i: (0, 0))]
        args += [wrow_n, wcol_n]
        out_specs = [pl.BlockSpec((BN, 64), lambda i: (i, 0)),
                     pl.BlockSpec((BN, 128), lambda i: (i, 0)),
                     pl.BlockSpec((BN, 128), lambda i: (i, 0))]
        out_shape = [jax.ShapeDtypeStruct((N, 64), jnp.float32),
                     jax.ShapeDtypeStruct((N, 128), jnp.float32),
                     jax.ShapeDtypeStruct((N, 128), jnp.float32)]
    return pl.pallas_call(
        functools.partial(_node_body, last),
        grid=(NB,),
        in_specs=in_specs,
        out_specs=out_specs,
        out_shape=out_shape,
    )(*args)


def _proj_body(x_ref, wrow_ref, wcol_ref, prow_ref, pcol_ref):
    xv = x_ref[...]
    prow_ref[...] = jnp.dot(xv, wrow_ref[...], precision=_HIGH,
                            preferred_element_type=jnp.float32)
    pcol_ref[...] = jnp.dot(xv, wcol_ref[...], precision=_HIGH,
                            preferred_element_type=jnp.float32)


def _project(x, wrow, wcol):
    dx = x.shape[1]
    return pl.pallas_call(
        _proj_body,
        grid=(NB,),
        in_specs=[
            pl.BlockSpec((BN, dx), lambda i: (i, 0)),
            pl.BlockSpec((dx, 128), lambda i: (0, 0)),
            pl.BlockSpec((dx, 128), lambda i: (0, 0)),
        ],
        out_specs=[pl.BlockSpec((BN, 128), lambda i: (i, 0)),
                   pl.BlockSpec((BN, 128), lambda i: (i, 0))],
        out_shape=[jax.ShapeDtypeStruct((N, 128), jnp.float32),
                   jax.ShapeDtypeStruct((N, 128), jnp.float32)],
    )(x, wrow, wcol)


def _split_params(p, dn):
    We, W1, W2 = p["We"], p["W1"], p["W2"]
    return dict(
        wes=We[:dn],
        wedp=jnp.concatenate(
            [We[dn:2 * dn], jnp.zeros((dn, 64), jnp.float32)], axis=1),
        wee=We[2 * dn:],
        w1x=W1[:dn], w1e=W1[dn:],
        w2x=W2[:dn], w2a=W2[dn:],
        be=p["be"][None, :], b1=p["b1"][None, :], b2=p["b2"][None, :],
    )


def kernel(x, edge_index, edge_attr, params):
    row1d = jnp.concatenate(
        [edge_index[0], jnp.zeros((E_PAD - E,), jnp.int32)])
    colg1d = jnp.concatenate(
        [edge_index[1], jnp.zeros((E_PAD - E,), jnp.int32)])
    colp1d = jnp.concatenate(
        [edge_index[1], jnp.full((E_PAD - E,), N_PAD - 1, jnp.int32)])

    ea = jnp.concatenate(
        [edge_attr, jnp.zeros((E_PAD - E, edge_attr.shape[1]),
                              jnp.float32)])
    zeros64 = jnp.zeros((N_PAD, 64), jnp.float32)
    ones64 = jnp.ones((CS, 64), jnp.float32)

    sp = [_split_params(p, 128 if i == 0 else 64)
          for i, p in enumerate(params)]

    cnt = _sc_count(colp1d, zeros64, ones64)

    wrow1 = jnp.concatenate([sp[0]["wes"], sp[0]["w1x"]], axis=1)
    prow, pcol = _project(x, wrow1, sp[0]["wedp"])

    for i in range(3):
        last = i == 2
        s = sp[i]
        grow, gcol = _sc_gather(prow, pcol, row1d, colg1d)
        e, m = _edge_mlp(grow, gcol, ea, s["wee"], s["w1e"],
                         s["be"], s["b1"], last)
        sums = _sc_scatter(m, colp1d, zeros64)
        if last:
            x = _node_update(x, sums, cnt, s["w2x"], s["w2a"], s["b2"],
                             None, None, True)
        else:
            sn = sp[i + 1]
            wrow_n = jnp.concatenate([sn["wes"], sn["w1x"]], axis=1)
            x, prow, pcol = _node_update(x, sums, cnt, s["w2x"], s["w2a"],
                                         s["b2"], wrow_n, sn["wedp"], False)
        ea = e
    return x, ea[:E]


# speedup vs baseline: 1.0665x; 1.0665x over previous
import functools

import jax
import jax.numpy as jnp
from jax import lax
from jax.experimental import pallas as pl
from jax.experimental.pallas import tpu as pltpu
from jax.experimental.pallas import tpu_sc as plsc

N = 10000
E = 320000
NC, NS = 2, 16
NW = NC * NS
E_PAD = 327680
EPW = E_PAD // NW
CG = 160
NG = EPW // CG
CS = 128
NCH = EPW // CS
N_PAD = 10240
NPT = N_PAD // NS
BE = 512
BN = 2048
NB = N_PAD // BN

_HIGH = jax.lax.Precision.HIGHEST


def _mesh():
    return plsc.VectorSubcoreMesh(core_axis_name="c", subcore_axis_name="s")


def _gather_body(prow_h, pcol_h, row_h, col_h, grow_h, gcol_h,
                 idx_r0, idx_r1, idx_c0, idx_c1,
                 buf_r0, buf_r1, buf_c0, buf_c1,
                 sem_gr0, sem_gr1, sem_gc0, sem_gc1,
                 sem_wr0, sem_wr1, sem_wc0, sem_wc1):
    wid = lax.axis_index("s") * NC + lax.axis_index("c")
    base = wid * EPW
    phases = (
        (idx_r0, idx_c0, buf_r0, buf_c0, sem_gr0, sem_gc0, sem_wr0, sem_wc0),
        (idx_r1, idx_c1, buf_r1, buf_c1, sem_gr1, sem_gc1, sem_wr1, sem_wc1),
    )

    def load_idx(j, ph):
        ir, ic = phases[ph][0], phases[ph][1]
        off = base + j * CG
        pltpu.sync_copy(row_h.at[pl.ds(off, CG)], ir)
        pltpu.sync_copy(col_h.at[pl.ds(off, CG)], ic)

    def start_gather(j, ph):
        ir, ic, br, bc, gr, gc = phases[ph][:6]
        pltpu.async_copy(prow_h.at[ir], br, gr)
        pltpu.async_copy(pcol_h.at[ic], bc, gc)

    def wait_gather(j, ph):
        ir, ic, br, bc, gr, gc = phases[ph][:6]
        pltpu.make_async_copy(prow_h.at[ir], br, gr).wait()
        pltpu.make_async_copy(pcol_h.at[ic], bc, gc).wait()

    def start_wb(j, ph):
        br, bc = phases[ph][2], phases[ph][3]
        wr, wc = phases[ph][6], phases[ph][7]
        off = base + j * CG
        pltpu.async_copy(br, grow_h.at[pl.ds(off, CG)], wr)
        pltpu.async_copy(bc, gcol_h.at[pl.ds(off, CG)], wc)

    def wait_wb(j, ph):
        br, bc = phases[ph][2], phases[ph][3]
        wr, wc = phases[ph][6], phases[ph][7]
        off = base + j * CG
        pltpu.make_async_copy(br, grow_h.at[pl.ds(off, CG)], wr).wait()
        pltpu.make_async_copy(bc, gcol_h.at[pl.ds(off, CG)], wc).wait()

    load_idx(0, 0)
    start_gather(0, 0)
    load_idx(1, 1)
    start_gather(1, 1)

    def body(k, carry):
        for ph in range(2):
            j = 2 * k + ph
            wait_gather(j, ph)
            start_wb(j, ph)

            @pl.when(j + 2 < NG)
            def _():
                load_idx(j + 2, ph)
            wait_wb(j, ph)

            @pl.when(j + 2 < NG)
            def _():
                start_gather(j + 2, ph)
        return carry

    lax.fori_loop(0, NG // 2, body, 0)


def _sc_gather(prow, pcol, row, col):
    return pl.kernel(
        _gather_body,
        out_type=[jax.ShapeDtypeStruct((E_PAD, 128), jnp.float32),
                  jax.ShapeDtypeStruct((E_PAD, 128), jnp.float32)],
        mesh=_mesh(),
        scratch_types=(
            [pltpu.VMEM((CG,), jnp.int32)] * 4
            + [pltpu.VMEM((CG, 128), jnp.float32)] * 4
            + [pltpu.SemaphoreType.DMA] * 8
        ),
    )(prow, pcol, row, col)


def _scatter_body(m_h, col_h, zero_h, sums_h, idx_a, idx_b, vals_a, vals_b,
                  acc_s):
    ci = lax.axis_index("c")
    si = lax.axis_index("s")
    wid = si * NC + ci
    pltpu.sync_copy(zero_h.at[pl.ds(si * NPT, NPT)],
                    acc_s.at[pl.ds(si * NPT, NPT)])
    plsc.subcore_barrier()

    def body(k, carry):
        for ph, ibuf, vbuf in ((0, idx_a, vals_a), (1, idx_b, vals_b)):
            j = 2 * k + ph
            off = wid * EPW + j * CS
            pltpu.sync_copy(col_h.at[pl.ds(off, CS)], ibuf)
            pltpu.sync_copy(m_h.at[pl.ds(off, CS)], vbuf)
            pltpu.sync_copy(vbuf, acc_s.at[ibuf], add=True)
        return carry

    lax.fori_loop(0, NCH // 2, body, 0)
    plsc.subcore_barrier()
    pltpu.sync_copy(acc_s.at[pl.ds(si * NPT, NPT)],
                    sums_h.at[pl.ds(ci * N_PAD + si * NPT, NPT)])


def _sc_scatter(m, colp, zeros64):
    return pl.kernel(
        _scatter_body,
        out_type=jax.ShapeDtypeStruct((NC * N_PAD, 64), jnp.float32),
        mesh=_mesh(),
        scratch_types=[
            pltpu.VMEM((CS,), jnp.int32),
            pltpu.VMEM((CS,), jnp.int32),
            pltpu.VMEM((CS, 64), jnp.float32),
            pltpu.VMEM((CS, 64), jnp.float32),
            pltpu.VMEM_SHARED((N_PAD, 64), jnp.float32),
        ],
    )(m, colp, zeros64)


def _count_body(col_h, zero_h, ones_h, cnt_h, idx_a, idx_b, ones_v, acc_s):
    ci = lax.axis_index("c")
    si = lax.axis_index("s")
    wid = si * NC + ci
    pltpu.sync_copy(zero_h.at[pl.ds(si * NPT, NPT)],
                    acc_s.at[pl.ds(si * NPT, NPT)])
    pltpu.sync_copy(ones_h, ones_v)
    plsc.subcore_barrier()

    def body(k, carry):
        for ph, ibuf in ((0, idx_a), (1, idx_b)):
            j = 2 * k + ph
            off = wid * EPW + j * CS
            pltpu.sync_copy(col_h.at[pl.ds(off, CS)], ibuf)
            pltpu.sync_copy(ones_v, acc_s.at[ibuf], add=True)
        return carry

    lax.fori_loop(0, NCH // 2, body, 0)
    plsc.subcore_barrier()
    pltpu.sync_copy(acc_s.at[pl.ds(si * NPT, NPT)],
                    cnt_h.at[pl.ds(ci * N_PAD + si * NPT, NPT)])


def _sc_count(colp, zeros64, ones64):
    return pl.kernel(
        _count_body,
        out_type=jax.ShapeDtypeStruct((NC * N_PAD, 64), jnp.float32),
        mesh=_mesh(),
        scratch_types=[
            pltpu.VMEM((CS,), jnp.int32),
            pltpu.VMEM((CS,), jnp.int32),
            pltpu.VMEM((CS, 64), jnp.float32),
            pltpu.VMEM_SHARED((N_PAD, 64), jnp.float32),
        ],
    )(colp, zeros64, ones64)


def _edge_body(last, g_ref, gc_ref, ea_ref, wee_ref, w1e_ref,
               be_ref, b1_ref, e_ref, m_ref):
    g = g_ref[...]
    e = (g[:, :64] + gc_ref[...][:, :64]
         + jnp.dot(ea_ref[...], wee_ref[...], precision=_HIGH,
                   preferred_element_type=jnp.float32)
         + be_ref[...])
    if not last:
        e = jnp.maximum(e, 0.0)
    e_ref[...] = e
    m = (g[:, 64:]
         + jnp.dot(e, w1e_ref[...], precision=_HIGH,
                   preferred_element_type=jnp.float32)
         + b1_ref[...])
    if not last:
        m = jnp.maximum(m, 0.0)
    m_ref[...] = m


def _edge_mlp(grow, gcol, ea, wee, w1e, be_, b1, last):
    de = ea.shape[1]
    return pl.pallas_call(
        functools.partial(_edge_body, last),
        grid=(E_PAD // BE,),
        in_specs=[
            pl.BlockSpec((BE, 128), lambda i: (i, 0)),
            pl.BlockSpec((BE, 128), lambda i: (i, 0)),
            pl.BlockSpec((BE, de), lambda i: (i, 0)),
            pl.BlockSpec((de, 64), lambda i: (0, 0)),
            pl.BlockSpec((64, 64), lambda i: (0, 0)),
            pl.BlockSpec((1, 64), lambda i: (0, 0)),
            pl.BlockSpec((1, 64), lambda i: (0, 0)),
        ],
        out_specs=[pl.BlockSpec((BE, 64), lambda i: (i, 0)),
                   pl.BlockSpec((BE, 64), lambda i: (i, 0))],
        out_shape=[jax.ShapeDtypeStruct((E_PAD, 64), jnp.float32),
                   jax.ShapeDtypeStruct((E_PAD, 64), jnp.float32)],
    )(grow, gcol, ea, wee, w1e, be_, b1)


def _node_body(last, x_ref, s0_ref, s1_ref, c0_ref, c1_ref, w2x_ref,
               w2a_ref, b2_ref, *rest):
    cnt = c0_ref[...][:, 0:1] + c1_ref[...][:, 0:1]
    recip = 1.0 / jnp.maximum(cnt, 1.0)
    agg = (s0_ref[...] + s1_ref[...]) * recip
    h = (jnp.dot(x_ref[...], w2x_ref[...], precision=_HIGH,
                 preferred_element_type=jnp.float32)
         + jnp.dot(agg, w2a_ref[...], precision=_HIGH,
                   preferred_element_type=jnp.float32)
         + b2_ref[...])
    if not last:
        h = jnp.maximum(h, 0.0)
    if last:
        (xn_ref,) = rest
        xn_ref[...] = h
    else:
        wrow_ref, wcol_ref, xn_ref, prow_ref, pcol_ref = rest
        xn_ref[...] = h
        prow_ref[...] = jnp.dot(h, wrow_ref[...], precision=_HIGH,
                                preferred_element_type=jnp.float32)
        pcol_ref[...] = jnp.dot(h, wcol_ref[...], precision=_HIGH,
                                preferred_element_type=jnp.float32)


def _node_update(x, sums, cnt, w2x, w2a, b2, wrow_n, wcol_n, last):
    dx = x.shape[1]
    in_specs = [
        pl.BlockSpec((BN, dx), lambda i: (i, 0)),
        pl.BlockSpec((BN, 64), lambda i: (i, 0)),
        pl.BlockSpec((BN, 64), lambda i: (i + NB, 0)),
        pl.BlockSpec((BN, 64), lambda i: (i, 0)),
        pl.BlockSpec((BN, 64), lambda i: (i + NB, 0)),
        pl.BlockSpec((dx, 64), lambda i: (0, 0)),
        pl.BlockSpec((64, 64), lambda i: (0, 0)),
        pl.BlockSpec((1, 64), lambda i: (0, 0)),
    ]
    args = [x, sums, sums, cnt, cnt, w2x, w2a, b2]
    if last:
        out_specs = pl.BlockSpec((BN, 64), lambda i: (i, 0))
        out_shape = jax.ShapeDtypeStruct((N, 64), jnp.float32)
    else:
        in_specs += [pl.BlockSpec((64, 128), lambda i: (0, 0)),
                     pl.BlockSpec((64, 128), lambda i: (0, 0))]
        args += [wrow_n, wcol_n]
        out_specs = [pl.BlockSpec((BN, 64), lambda i: (i, 0)),
                     pl.BlockSpec((BN, 128), lambda i: (i, 0)),
                     pl.BlockSpec((BN, 128), lambda i: (i, 0))]
        out_shape = [jax.ShapeDtypeStruct((N, 64), jnp.float32),
                     jax.ShapeDtypeStruct((N, 128), jnp.float32),
                     jax.ShapeDtypeStruct((N, 128), jnp.float32)]
    return pl.pallas_call(
        functools.partial(_node_body, last),
        grid=(NB,),
        in_specs=in_specs,
        out_specs=out_specs,
        out_shape=out_shape,
    )(*args)


def _proj_body(x_ref, wrow_ref, wcol_ref, prow_ref, pcol_ref):
    xv = x_ref[...]
    prow_ref[...] = jnp.dot(xv, wrow_ref[...], precision=_HIGH,
                            preferred_element_type=jnp.float32)
    pcol_ref[...] = jnp.dot(xv, wcol_ref[...], precision=_HIGH,
                            preferred_element_type=jnp.float32)


def _project(x, wrow, wcol):
    dx = x.shape[1]
    return pl.pallas_call(
        _proj_body,
        grid=(NB,),
        in_specs=[
            pl.BlockSpec((BN, dx), lambda i: (i, 0)),
            pl.BlockSpec((dx, 128), lambda i: (0, 0)),
            pl.BlockSpec((dx, 128), lambda i: (0, 0)),
        ],
        out_specs=[pl.BlockSpec((BN, 128), lambda i: (i, 0)),
                   pl.BlockSpec((BN, 128), lambda i: (i, 0))],
        out_shape=[jax.ShapeDtypeStruct((N, 128), jnp.float32),
                   jax.ShapeDtypeStruct((N, 128), jnp.float32)],
    )(x, wrow, wcol)


def _split_params(p, dn):
    We, W1, W2 = p["We"], p["W1"], p["W2"]
    return dict(
        wes=We[:dn],
        wedp=jnp.concatenate(
            [We[dn:2 * dn], jnp.zeros((dn, 64), jnp.float32)], axis=1),
        wee=We[2 * dn:],
        w1x=W1[:dn], w1e=W1[dn:],
        w2x=W2[:dn], w2a=W2[dn:],
        be=p["be"][None, :], b1=p["b1"][None, :], b2=p["b2"][None, :],
    )


def kernel(x, edge_index, edge_attr, params):
    row1d = jnp.concatenate(
        [edge_index[0], jnp.zeros((E_PAD - E,), jnp.int32)])
    colg1d = jnp.concatenate(
        [edge_index[1], jnp.zeros((E_PAD - E,), jnp.int32)])
    colp1d = jnp.concatenate(
        [edge_index[1], jnp.full((E_PAD - E,), N_PAD - 1, jnp.int32)])

    ea = jnp.concatenate(
        [edge_attr, jnp.zeros((E_PAD - E, edge_attr.shape[1]),
                              jnp.float32)])
    zeros64 = jnp.zeros((N_PAD, 64), jnp.float32)
    ones64 = jnp.ones((CS, 64), jnp.float32)

    sp = [_split_params(p, 128 if i == 0 else 64)
          for i, p in enumerate(params)]

    cnt = _sc_count(colp1d, zeros64, ones64)

    wrow1 = jnp.concatenate([sp[0]["wes"], sp[0]["w1x"]], axis=1)
    prow, pcol = _project(x, wrow1, sp[0]["wedp"])

    for i in range(3):
        last = i == 2
        s = sp[i]
        grow, gcol = _sc_gather(prow, pcol, row1d, colg1d)
        e, m = _edge_mlp(grow, gcol, ea, s["wee"], s["w1e"],
                         s["be"], s["b1"], last)
        sums = _sc_scatter(m, colp1d, zeros64)
        if last:
            x = _node_update(x, sums, cnt, s["w2x"], s["w2a"], s["b2"],
                             None, None, True)
        else:
            sn = sp[i + 1]
            wrow_n = jnp.concatenate([sn["wes"], sn["w1x"]], axis=1)
            x, prow, pcol = _node_update(x, sums, cnt, s["w2x"], s["w2a"],
                                         s["b2"], wrow_n, sn["wedp"], False)
        ea = e
    return x, ea[:E]
